# Initial kernel scaffold; baseline (speedup 1.0000x reference)
import jax
import jax.numpy as jnp
from jax.experimental import pallas as pl


def kernel(x, edge_index, batch, W1, b1, W2, b2, fW1, fb1, fW2, fb2, sW1, sb1, sW2, sb2, dW1, db1, dW2, db2):
    raise NotImplementedError("write your pallas kernel here")


# speedup vs baseline: 8.4816x; 8.4816x over previous
import functools

import jax
import jax.numpy as jnp
from jax import lax
from jax.experimental import pallas as pl
from jax.experimental.pallas import tpu as pltpu
from jax.experimental.pallas import tpu_sc as plsc

N = 10000
E = 320000
D = 128
H = 128
NPAD = 10240
NC = 2
NS = 16
NW = NC * NS
EPW = E // NW
CH = 80
NCHUNK = EPW // CH
RPW = NPAD // NS

def _mesh():
    return plsc.VectorSubcoreMesh(
        core_axis_name="c", subcore_axis_name="s",
        num_cores=NC, num_subcores=NS)


def _worker():
    cid = lax.axis_index("c")
    sid = lax.axis_index("s")
    return cid, sid, cid * NS + sid


@functools.cache
def _make_deg(W=128):
    @functools.partial(
        pl.kernel,
        out_type=jax.ShapeDtypeStruct((NC, NPAD, W), jnp.float32),
        mesh=_mesh(),
        scratch_types=[
            pltpu.VMEM((CH,), jnp.int32),
            pltpu.VMEM((CH, W), jnp.float32),
            pltpu.VMEM_SHARED((NPAD, W), jnp.float32),
        ],
    )
    def deg_kernel(dst_hbm, zeros_hbm, ones_hbm, out_hbm, idx_v, ones_v,
                   acc_sh):
        cid, sid, wid = _worker()
        pltpu.sync_copy(zeros_hbm.at[pl.ds(sid * RPW, RPW), :],
                        acc_sh.at[pl.ds(sid * RPW, RPW), :])
        pltpu.sync_copy(ones_hbm, ones_v)
        plsc.subcore_barrier()
        base = wid * EPW

        @pl.loop(0, NCHUNK)
        def _chunks(j):
            pltpu.sync_copy(dst_hbm.at[pl.ds(base + j * CH, CH)], idx_v)
            pltpu.sync_copy(ones_v, acc_sh.at[idx_v], add=True)

        plsc.subcore_barrier()
        pltpu.sync_copy(acc_sh.at[pl.ds(sid * RPW, RPW), :],
                        out_hbm.at[cid, pl.ds(sid * RPW, RPW), :])

    return deg_kernel


@functools.cache
def _make_prop():
    @functools.partial(
        pl.kernel,
        out_type=jax.ShapeDtypeStruct((NC, NPAD, H), jnp.float32),
        mesh=_mesh(),
        scratch_types=[
            pltpu.VMEM((CH,), jnp.int32),
            pltpu.VMEM((CH,), jnp.int32),
            pltpu.VMEM((CH, H), jnp.float32),
            pltpu.VMEM_SHARED((NPAD, H), jnp.float32),
            pltpu.SemaphoreType.DMA,
        ],
    )
    def prop_kernel(y_hbm, src_hbm, dst_hbm, zeros_hbm, out_hbm,
                    idx_s, idx_d, rows, acc_sh, sem):
        cid, sid, wid = _worker()
        pltpu.sync_copy(zeros_hbm.at[pl.ds(sid * RPW, RPW), :],
                        acc_sh.at[pl.ds(sid * RPW, RPW), :])
        plsc.subcore_barrier()
        base = wid * EPW

        @pl.loop(0, NCHUNK)
        def _chunks(j):
            pltpu.sync_copy(src_hbm.at[pl.ds(base + j * CH, CH)], idx_s)
            pltpu.sync_copy(dst_hbm.at[pl.ds(base + j * CH, CH)], idx_d)
            pltpu.async_copy(y_hbm.at[idx_s], rows, sem).wait()
            pltpu.sync_copy(rows, acc_sh.at[idx_d], add=True)

        plsc.subcore_barrier()
        pltpu.sync_copy(acc_sh.at[pl.ds(sid * RPW, RPW), :],
                        out_hbm.at[cid, pl.ds(sid * RPW, RPW), :])

    return prop_kernel


@functools.cache
def _make_struct():
    @functools.partial(
        pl.kernel,
        out_type=jax.ShapeDtypeStruct((E,), jnp.float32),
        mesh=_mesh(),
        compiler_params=pltpu.CompilerParams(needs_layout_passes=False),
        scratch_types=[
            pltpu.VMEM((CH,), jnp.int32),
            pltpu.VMEM((CH,), jnp.int32),
            pltpu.VMEM((CH, H), jnp.float32),
            pltpu.VMEM((CH, H), jnp.float32),
            pltpu.VMEM((H,), jnp.float32),
            pltpu.VMEM((16,), jnp.float32),
            pltpu.VMEM((CH,), jnp.float32),
            pltpu.VMEM((256,), jnp.float32),
            pltpu.SemaphoreType.DMA,
            pltpu.SemaphoreType.DMA,
        ],
    )
    def struct_kernel(a_hbm, b_hbm, src_hbm, dst_hbm, w2_hbm, sb2_hbm,
                      out_hbm, idx_s, idx_d, rows_a, rows_b, w2_v, sb2_v,
                      dot_v, tbuf, sem_a, sem_b):
        cid, sid, wid = _worker()
        pltpu.sync_copy(w2_hbm, w2_v)
        pltpu.sync_copy(sb2_hbm, sb2_v)
        base = wid * EPW

        @pl.loop(0, NCHUNK)
        def _chunks(j):
            pltpu.sync_copy(src_hbm.at[pl.ds(base + j * CH, CH)], idx_s)
            pltpu.sync_copy(dst_hbm.at[pl.ds(base + j * CH, CH)], idx_d)
            cp_a = pltpu.async_copy(a_hbm.at[idx_s], rows_a, sem_a)
            cp_b = pltpu.async_copy(b_hbm.at[idx_d], rows_b, sem_b)
            cp_a.wait()
            cp_b.wait()

            @pl.loop(0, CH // 16)
            def _groups(g):
                for e16 in range(16):
                    e = g * 16 + e16
                    acc = jnp.zeros((16,), jnp.float32)
                    for c in range(H // 16):
                        va = rows_a[e, pl.ds(c * 16, 16)]
                        vb = rows_b[e, pl.ds(c * 16, 16)]
                        t = jnp.maximum(va + vb, 0.0)
                        acc = acc + t * w2_v[pl.ds(c * 16, 16)]
                    tbuf[pl.ds(e16 * 16, 16)] = acc
                flat = lax.iota(jnp.int32, 16) * 16
                vsum = jnp.zeros((16,), jnp.float32)
                for c in range(16):
                    vsum = vsum + plsc.load_gather(tbuf, [flat + c])
                t = vsum + sb2_v[...]
                dot_v[pl.ds(g * 16, 16)] = 1.0 / (1.0 + jnp.exp(-t))

            pltpu.sync_copy(dot_v, out_hbm.at[pl.ds(base + j * CH, CH)])

    return struct_kernel


def _mm1_body(x_ref, w_ref, deg_ref, y_ref, dis_ref):
    dis = lax.rsqrt(deg_ref[...])
    xw = jnp.dot(x_ref[...], w_ref[...], preferred_element_type=jnp.float32,
                 precision=lax.Precision.HIGHEST)
    y_ref[...] = xw * dis
    dis_ref[...] = dis


def _mm1(xp, W1, deg2):
    R = 1024
    return pl.pallas_call(
        _mm1_body,
        grid=(NPAD // R,),
        in_specs=[
            pl.BlockSpec((R, D), lambda i: (i, 0)),
            pl.BlockSpec((D, H), lambda i: (0, 0)),
            pl.BlockSpec((R, 1), lambda i: (i, 0)),
        ],
        out_specs=[
            pl.BlockSpec((R, H), lambda i: (i, 0)),
            pl.BlockSpec((R, 1), lambda i: (i, 0)),
        ],
        out_shape=[
            jax.ShapeDtypeStruct((NPAD, H), jnp.float32),
            jax.ShapeDtypeStruct((NPAD, 1), jnp.float32),
        ],
    )(xp, W1, deg2)


def _mm2_body(acc_ref, y1_ref, dis_ref, b1_ref, w2_ref, y2_ref):
    dis = dis_ref[...]
    s = acc_ref[0] + acc_ref[1] + y1_ref[...]
    h = jnp.maximum(dis * s + b1_ref[...], 0.0)
    y2_ref[...] = jnp.dot(h, w2_ref[...],
                          preferred_element_type=jnp.float32,
                 precision=lax.Precision.HIGHEST) * dis


def _mm2(acc1, y1, dis, b1, W2):
    R = 1024
    return pl.pallas_call(
        _mm2_body,
        grid=(NPAD // R,),
        in_specs=[
            pl.BlockSpec((NC, R, H), lambda i: (0, i, 0)),
            pl.BlockSpec((R, H), lambda i: (i, 0)),
            pl.BlockSpec((R, 1), lambda i: (i, 0)),
            pl.BlockSpec((1, H), lambda i: (0, 0)),
            pl.BlockSpec((H, H), lambda i: (0, 0)),
        ],
        out_specs=pl.BlockSpec((R, H), lambda i: (i, 0)),
        out_shape=jax.ShapeDtypeStruct((NPAD, H), jnp.float32),
    )(acc1, y1, dis, b1, W2)


def _mm3_body(acc_ref, y2_ref, dis_ref, b2_ref, fw1_ref, fb1_ref, fw2_ref,
              fb2_ref, sw1a_ref, sw1b_ref, sb1_ref, dw1_ref, db1_ref,
              dw2t_ref, db2_ref, z_ref, fr_ref, a_ref, bb_ref, pd_ref):
    dis = dis_ref[...]
    s = acc_ref[0] + acc_ref[1] + y2_ref[...]
    z = dis * s + b2_ref[...]
    z_ref[...] = z
    t = jnp.dot(z, fw1_ref[...], preferred_element_type=jnp.float32,
                 precision=lax.Precision.HIGHEST) \
        + fb1_ref[...]
    f = jnp.where(t > 0, t, 0.1 * t)
    fr_ref[...] = jnp.dot(f, fw2_ref[...],
                          preferred_element_type=jnp.float32,
                 precision=lax.Precision.HIGHEST) + fb2_ref[...]
    a_ref[...] = jnp.dot(z, sw1a_ref[...],
                         preferred_element_type=jnp.float32,
                 precision=lax.Precision.HIGHEST) + sb1_ref[...]
    bb_ref[...] = jnp.dot(z, sw1b_ref[...],
                          preferred_element_type=jnp.float32,
                 precision=lax.Precision.HIGHEST)
    dh = jnp.maximum(jnp.dot(z, dw1_ref[...],
                             preferred_element_type=jnp.float32,
                 precision=lax.Precision.HIGHEST)
                     + db1_ref[...], 0.0)
    pd_ref[...] = jnp.sum(dh * dw2t_ref[...], axis=1, keepdims=True) \
        + db2_ref[...]


def _mm3(acc2, y2, dis, b2, fW1, fb1, fW2, fb2, sW1a, sW1b, sb1, dW1, db1,
         dW2t, db2):
    R = 1024
    return pl.pallas_call(
        _mm3_body,
        grid=(NPAD // R,),
        in_specs=[
            pl.BlockSpec((NC, R, H), lambda i: (0, i, 0)),
            pl.BlockSpec((R, H), lambda i: (i, 0)),
            pl.BlockSpec((R, 1), lambda i: (i, 0)),
            pl.BlockSpec((1, H), lambda i: (0, 0)),
            pl.BlockSpec((H, H // 2), lambda i: (0, 0)),
            pl.BlockSpec((1, H // 2), lambda i: (0, 0)),
            pl.BlockSpec((H // 2, D), lambda i: (0, 0)),
            pl.BlockSpec((1, D), lambda i: (0, 0)),
            pl.BlockSpec((H, H), lambda i: (0, 0)),
            pl.BlockSpec((H, H), lambda i: (0, 0)),
            pl.BlockSpec((1, H), lambda i: (0, 0)),
            pl.BlockSpec((H, H), lambda i: (0, 0)),
            pl.BlockSpec((1, H), lambda i: (0, 0)),
            pl.BlockSpec((1, H), lambda i: (0, 0)),
            pl.BlockSpec((1, 1), lambda i: (0, 0)),
        ],
        out_specs=[
            pl.BlockSpec((R, H), lambda i: (i, 0)),
            pl.BlockSpec((R, D), lambda i: (i, 0)),
            pl.BlockSpec((R, H), lambda i: (i, 0)),
            pl.BlockSpec((R, H), lambda i: (i, 0)),
            pl.BlockSpec((R, 1), lambda i: (i, 0)),
        ],
        out_shape=[
            jax.ShapeDtypeStruct((NPAD, H), jnp.float32),
            jax.ShapeDtypeStruct((NPAD, D), jnp.float32),
            jax.ShapeDtypeStruct((NPAD, H), jnp.float32),
            jax.ShapeDtypeStruct((NPAD, H), jnp.float32),
            jax.ShapeDtypeStruct((NPAD, 1), jnp.float32),
        ],
    )(acc2, y2, dis, b2, fW1, fb1, fW2, fb2, sW1a, sW1b, sb1, dW1, db1,
      dW2t, db2)


def kernel(x, edge_index, batch, W1, b1, W2, b2, fW1, fb1, fW2, fb2,
           sW1, sb1, sW2, sb2, dW1, db1, dW2, db2):
    src = edge_index[0]
    dst = edge_index[1]
    xp = jnp.pad(x, ((0, NPAD - N), (0, 0)))

    zerosH = jnp.zeros((NPAD, H), jnp.float32)
    onesH = jnp.ones((CH, H), jnp.float32)

    degp = _make_deg()(dst, zerosH, onesH)
    deg2 = (degp[0, :, :1] + degp[1, :, :1]) + 1.0

    y1, dis = _mm1(xp, W1, deg2)
    acc1 = _make_prop()(y1, src, dst, zerosH)
    y2 = _mm2(acc1, y1, dis, b1.reshape(1, H), W2)
    acc2 = _make_prop()(y2, src, dst, zerosH)

    z, fr, A, B, pd = _mm3(
        acc2, y2, dis, b2.reshape(1, H),
        fW1, fb1.reshape(1, H // 2), fW2, fb2.reshape(1, D),
        sW1[:H], sW1[H:], sb1.reshape(1, H),
        dW1, db1.reshape(1, H), dW2.reshape(1, H), db2.reshape(1, 1))

    sb2v = jnp.full((16,), sb2[0], jnp.float32)
    sr = _make_struct()(A, B, src, dst, sW2.reshape(H), sb2v)

    return (z[:N], fr[:N], sr.reshape(E, 1), pd[:N])
